# Initial kernel scaffold; baseline (speedup 1.0000x reference)
import jax
import jax.numpy as jnp
from jax.experimental import pallas as pl


def kernel(post_activation_yx, center_yx, img_idxs, head_idxs, grid_y_idxs, grid_x_idxs, object_idxs):
    raise NotImplementedError("write your pallas kernel here")


# speedup vs baseline: 6.1486x; 6.1486x over previous
import functools

import jax
import jax.numpy as jnp
from jax import lax
from jax.experimental import pallas as pl
from jax.experimental.pallas import tpu as pltpu
from jax.experimental.pallas import tpu_sc as plsc

NC = 2
NS = 16
NW = NC * NS
L = 16

CHUNK = 2048
NCHUNK = 8
PER_TILE = CHUNK * NCHUNK


def _make_sc_loss(n, nobj2, s_img, s_head, s_y, plane):

    @functools.partial(
        pl.kernel,
        out_type=jax.ShapeDtypeStruct((NW, L), jnp.float32),
        mesh=plsc.VectorSubcoreMesh(core_axis_name="c", subcore_axis_name="s"),
        compiler_params=pltpu.CompilerParams(needs_layout_passes=False),
        scratch_types=[
            pltpu.VMEM((nobj2,), jnp.float32),
            pltpu.VMEM((CHUNK,), jnp.int32),
            pltpu.VMEM((CHUNK,), jnp.int32),
            pltpu.VMEM((CHUNK,), jnp.int32),
            pltpu.VMEM((CHUNK,), jnp.int32),
            pltpu.VMEM((CHUNK,), jnp.int32),
            pltpu.VMEM((CHUNK,), jnp.int32),
            pltpu.VMEM((CHUNK,), jnp.int32),
            pltpu.VMEM((CHUNK,), jnp.int32),
            pltpu.VMEM((CHUNK,), jnp.int32),
            pltpu.VMEM((CHUNK,), jnp.int32),
            pltpu.VMEM((CHUNK,), jnp.float32),
            pltpu.VMEM((CHUNK,), jnp.float32),
            pltpu.VMEM((CHUNK,), jnp.float32),
            pltpu.VMEM((CHUNK,), jnp.float32),
            pltpu.VMEM((L,), jnp.float32),
            pltpu.SemaphoreType.DMA,
            pltpu.SemaphoreType.DMA,
        ],
    )
    def sc_loss(pa_hbm, cent_hbm, img_hbm, head_hbm, gy_hbm, gx_hbm,
                obj_hbm, out_hbm, cent_v, img_v, head_v, gy_v, gx_v,
                obj0_v, obj1_v, idxy0_v, idxy1_v, idxx0_v, idxx1_v,
                predy0_v, predy1_v, predx0_v, predx1_v, acc_v, sem0, sem1):
        wid = lax.axis_index("s") * NC + lax.axis_index("c")
        base = wid * PER_TILE
        sems = (sem0, sem1)
        objs = (obj0_v, obj1_v)
        idxys = (idxy0_v, idxy1_v)
        idxxs = (idxx0_v, idxx1_v)
        predys = (predy0_v, predy1_v)
        predxs = (predx0_v, predx1_v)
        pltpu.sync_copy(cent_hbm, cent_v)

        def stage(ci):
            p = ci & 1
            off = base + ci * CHUNK
            off_c = jnp.minimum(off, n - CHUNK)
            pltpu.sync_copy(img_hbm.at[pl.ds(off_c, CHUNK)], img_v)
            pltpu.sync_copy(head_hbm.at[pl.ds(off_c, CHUNK)], head_v)
            pltpu.sync_copy(gy_hbm.at[pl.ds(off_c, CHUNK)], gy_v)
            pltpu.sync_copy(gx_hbm.at[pl.ds(off_c, CHUNK)], gx_v)
            pltpu.sync_copy(obj_hbm.at[pl.ds(off_c, CHUNK)], objs[p])

            def ix_body(j, _):
                s = pl.ds(j * L, L)
                fy = (img_v[s] * s_img + head_v[s] * s_head
                      + gy_v[s] * s_y + gx_v[s])
                idxys[p][s] = fy
                idxxs[p][s] = fy + plane
                return 0

            lax.fori_loop(0, CHUNK // L, ix_body, 0, unroll=4)
            cpy = pltpu.async_copy(pa_hbm.at[idxys[p]], predys[p], sems[p])
            cpx = pltpu.async_copy(pa_hbm.at[idxxs[p]], predxs[p], sems[p])
            return cpy, cpx

        def consume(ci, cpy, cpx, acc):
            p = ci & 1
            off = base + ci * CHUNK
            off_c = jnp.minimum(off, n - CHUNK)
            cpy.wait()
            cpx.wait()
            lane = lax.iota(jnp.int32, L) + (off_c - off)

            def loss_body(j, a):
                s = pl.ds(j * L, L)
                ob2 = objs[p][s] * 2
                cy = plsc.load_gather(cent_v, [ob2])
                cx = plsc.load_gather(cent_v, [ob2 + 1])
                dy = predys[p][s] - cy
                dx = predxs[p][s] - cx
                contrib = dy * dy + dx * dx
                keep = (lane + j * L) >= 0
                return a + jnp.where(keep, contrib, 0.0)

            return lax.fori_loop(0, CHUNK // L, loss_body, acc, unroll=4)

        acc = jnp.zeros((L,), jnp.float32)
        cps = stage(0)
        for ci in range(NCHUNK):
            nxt = stage(ci + 1) if ci + 1 < NCHUNK else None
            acc = consume(ci, *cps, acc)
            cps = nxt

        acc_v[...] = acc
        pltpu.sync_copy(acc_v, out_hbm.at[wid])

    return sc_loss


def kernel(post_activation_yx, center_yx, img_idxs, head_idxs,
           grid_y_idxs, grid_x_idxs, object_idxs):
    b, nh, two, gy, gx = post_activation_yx.shape
    n = img_idxs.shape[0]
    nobj = center_yx.shape[0]
    assert two == 2 and n <= NW * PER_TILE and n >= CHUNK and n % 16 == 0

    plane = gy * gx
    s_head = two * plane
    s_img = nh * s_head
    s_y = gx

    pa_flat = post_activation_yx.reshape(-1)
    cent_flat = center_yx.reshape(-1)
    sc_loss = _make_sc_loss(n, nobj * 2, s_img, s_head, s_y, plane)
    partials = sc_loss(
        pa_flat, cent_flat,
        img_idxs.astype(jnp.int32), head_idxs.astype(jnp.int32),
        grid_y_idxs.astype(jnp.int32), grid_x_idxs.astype(jnp.int32),
        object_idxs.astype(jnp.int32),
    )
    return jnp.sum(partials)
